# Initial kernel scaffold; baseline (speedup 1.0000x reference)
import jax
import jax.numpy as jnp
from jax.experimental import pallas as pl


def kernel(data, weights):
    raise NotImplementedError("write your pallas kernel here")


# speedup vs baseline: 2.2407x; 2.2407x over previous
import jax
import jax.numpy as jnp
from jax import lax
from jax.experimental import pallas as pl
from jax.experimental.pallas import tpu as pltpu
from jax.experimental.pallas import tpu_sc as plsc

_Q = 250000
_P = 12
_NWMAX = 7
_THETA = 20.0
_NW = 32
_NG = 489
_RPT = _NG * 16
_LAST = _Q - _RPT
_TAILROWS = _Q - 31 * _RPT
_FILLW = _RPT * _P
_TAILW = _TAILROWS * _P
_EXTW = _FILLW - _TAILW
_INF = float("inf")

_mesh = plsc.VectorSubcoreMesh(core_axis_name="c", subcore_axis_name="s")
_params = pltpu.CompilerParams(needs_layout_passes=False)


def _wid():
    return lax.axis_index("c") * 16 + lax.axis_index("s")


def _splat(x, dtype=jnp.float32):
    return jnp.full((16,), x, dtype=dtype)


def _k1_body(inp16_hbm, w_hbm, inpout_hbm, pf_hbm, pi_hbm,
             vm_inp, vm_perm, vm_w, vm_fill, vm_pf, vm_pi,
             sem_w, sem_f):
    wid = _wid()
    start = jnp.minimum(wid * _RPT, _LAST)
    wofs = pl.multiple_of(start * _P, 8)
    wcopy = pltpu.make_async_copy(w_hbm.at[pl.ds(wofs, _FILLW)], vm_w, sem_w)
    wcopy.start()

    pltpu.sync_copy(inp16_hbm, vm_inp)
    v = vm_inp[...]
    iota = lax.iota(jnp.int32, 16)
    one = jnp.ones((16,), jnp.int32)
    zero = jnp.zeros((16,), jnp.int32)
    r = zero
    for k in range(_P):
        vk = _splat(v[k])
        r = r + jnp.where(vk < v, one, zero)
        r = r + jnp.where((vk == v) & (iota > k), one, zero)
    r = jnp.where(iota < _P, r, iota)
    plsc.store_scatter(vm_perm, [r], iota)
    pv = vm_perm[...]
    st = plsc.load_gather(vm_inp, [pv])

    pat = [plsc.load_gather(vm_inp, [(iota + 16 * k) % _P]) for k in range(3)]

    def fill_body(j, c):
        base = j * 48
        vm_fill[pl.ds(base, 16)] = pat[0]
        vm_fill[pl.ds(base + 16, 16)] = pat[1]
        vm_fill[pl.ds(base + 32, 16)] = pat[2]
        return c

    lax.fori_loop(0, 256, fill_body, 0)

    fills = []
    for k in range(8):
        s_k = min(k * 1024, _RPT - 1024)
        ofs = pl.multiple_of((start + s_k) * _P, 8)
        c = pltpu.make_async_copy(vm_fill, inpout_hbm.at[pl.ds(ofs, 12288)], sem_f)
        c.start()
        fills.append(c)

    perm_s = [pv[p] for p in range(_P)]
    st_s = [st[p] for p in range(_P)]
    c1v = [_splat(1.0 - st_s[p]) for p in range(_P)]
    emask = [_splat(st_s[p]) == _INF for p in range(_P)]
    liota12 = iota * _P

    wcopy.wait()

    def group(g, carry):
        m, s, i = carry
        b = g * (16 * _P)
        wr = [plsc.load_gather(vm_w, [liota12 + (b + perm_s[p])])
              for p in range(_P)]
        acc = jnp.where(emask[0], 0.0, wr[0])
        acc0 = acc
        found = acc >= _THETA
        ec = jnp.where(found, _splat(st_s[0]), _INF)
        pot = jnp.where(found, acc, 0.0)
        for p in range(1, _P):
            acc = acc + jnp.where(emask[p], 0.0, wr[p])
            cross = (acc >= _THETA) & (~found)
            ec = jnp.where(cross, _splat(st_s[p]), ec)
            pot = jnp.where(cross, acc, pot)
            found = found | cross
        u = [ec + c1v[p] for p in range(_P)]
        potr = None
        ind2 = jnp.zeros((16,), jnp.float32)
        f2 = None
        for w in range(_NWMAX):
            wf = jnp.float32(w)
            rs0 = jnp.maximum(jnp.minimum(u[0] + wf, wr[0]), 0.0)
            rs1 = jnp.maximum(jnp.minimum(u[1] + wf, wr[1]), 0.0)
            rs2 = jnp.maximum(jnp.minimum(u[2] + wf, wr[2]), 0.0)
            for p in range(3, _P, 3):
                rs0 = rs0 + jnp.maximum(jnp.minimum(u[p] + wf, wr[p]), 0.0)
                rs1 = rs1 + jnp.maximum(jnp.minimum(u[p + 1] + wf, wr[p + 1]), 0.0)
                rs2 = rs2 + jnp.maximum(jnp.minimum(u[p + 2] + wf, wr[p + 2]), 0.0)
            rs = rs0 + rs1 + rs2
            binr = rs >= _THETA
            if w == 0:
                potr = rs
                f2 = binr
            else:
                sel = binr & (~f2)
                potr = jnp.where(sel, rs, potr)
                ind2 = jnp.where(sel, wf, ind2)
                f2 = f2 | sel
        ec_f = jnp.where(found, ec + ind2, _INF)
        pot_f = jnp.where(found, potr, acc0)
        row = iota + (start + g * 16)
        lt = ec_f < m
        upd = lt | ((ec_f == m) & (pot_f > s))
        m = jnp.where(lt, ec_f, m)
        s = jnp.where(upd, pot_f, s)
        i = jnp.where(upd, row, i)
        return m, s, i

    init = (_splat(_INF), _splat(-_INF), jnp.zeros((16,), jnp.int32))
    m, s, i = lax.fori_loop(0, _NG, group, init)

    vm_pf[pl.ds(0, 16)] = m
    vm_pf[pl.ds(16, 16)] = s
    vm_pi[...] = i
    pltpu.sync_copy(vm_pf, pf_hbm.at[pl.ds(pl.multiple_of(wid * 32, 8), 32)])
    pltpu.sync_copy(vm_pi, pi_hbm.at[pl.ds(pl.multiple_of(wid * 16, 8), 16)])
    for c in fills:
        c.wait()


def _k2_body(pf_hbm, pi_hbm, stdp_hbm, next_hbm,
             vm_pf, vm_pi, vm_inf, vm_win, vm_idx, sem):
    wid = _wid()
    pltpu.sync_copy(pf_hbm, vm_pf)
    pltpu.sync_copy(pi_hbm, vm_pi)

    m = vm_pf[pl.ds(0, 16)]
    s = vm_pf[pl.ds(16, 16)]
    i = vm_pi[pl.ds(0, 16)]
    for t in range(1, _NW):
        mt = vm_pf[pl.ds(t * 32, 16)]
        st_ = vm_pf[pl.ds(t * 32 + 16, 16)]
        it = vm_pi[pl.ds(t * 16, 16)]
        take = (mt < m) | ((mt == m) & ((st_ > s) | ((st_ == s) & (it < i))))
        m = jnp.where(take, mt, m)
        s = jnp.where(take, st_, s)
        i = jnp.where(take, it, i)
    minv = jnp.min(m)
    mask = m == minv
    sm = jnp.where(mask, s, -_INF)
    smax = jnp.max(sm)
    ibig = jnp.where(mask & (sm == smax), i, jnp.int32(2**31 - 1))
    iid = jnp.min(ibig)

    infv = _splat(_INF)

    def fill_body(j, c):
        vm_inf[pl.ds(j * 16, 16)] = infv
        return c

    lax.fori_loop(0, _TAILW // 16, fill_body, 0)

    start = wid * _RPT
    pltpu.sync_copy(vm_inf,
                    stdp_hbm.at[pl.ds(pl.multiple_of(start * _P, 8), _TAILW)])
    pltpu.sync_copy(vm_inf.at[pl.ds(0, _TAILROWS)],
                    next_hbm.at[pl.ds(pl.multiple_of(start, 8), _TAILROWS)])

    @pl.when(wid < _NW - 1)
    def _extra():
        pltpu.sync_copy(
            vm_inf.at[pl.ds(0, _EXTW)],
            stdp_hbm.at[pl.ds(pl.multiple_of(start * _P + _TAILW, 8), _EXTW)])
        pltpu.sync_copy(
            vm_inf.at[pl.ds(0, _RPT - _TAILROWS)],
            next_hbm.at[pl.ds(pl.multiple_of(start + _TAILROWS, 8),
                              _RPT - _TAILROWS)])

    span = jnp.minimum(start + _RPT, _Q)

    @pl.when((iid >= start) & (iid < span))
    def _scatter():
        vm_win[...] = _splat(minv)
        offs = jnp.where(lax.iota(jnp.int32, 16) < _P,
                         lax.iota(jnp.int32, 16), 0)
        vm_idx[...] = offs + iid * _P
        pltpu.async_copy(vm_win, stdp_hbm.at[vm_idx], sem).wait()
        vm_idx[...] = _splat(iid, jnp.int32)
        pltpu.async_copy(vm_win, next_hbm.at[vm_idx], sem).wait()


@jax.jit
def _run(inp16, wflat):
    k1 = pl.kernel(
        _k1_body,
        out_type=[
            jax.ShapeDtypeStruct((_Q * _P,), jnp.float32),
            jax.ShapeDtypeStruct((_NW * 32,), jnp.float32),
            jax.ShapeDtypeStruct((_NW * 16,), jnp.int32),
        ],
        mesh=_mesh,
        scratch_types=[
            pltpu.VMEM((16,), jnp.float32),
            pltpu.VMEM((16,), jnp.int32),
            pltpu.VMEM((_FILLW,), jnp.float32),
            pltpu.VMEM((12288,), jnp.float32),
            pltpu.VMEM((32,), jnp.float32),
            pltpu.VMEM((16,), jnp.int32),
            pltpu.SemaphoreType.DMA,
            pltpu.SemaphoreType.DMA,
        ],
        compiler_params=_params,
        name="tnn_k1",
    )
    inp_out, pf, pi = k1(inp16, wflat)
    k2 = pl.kernel(
        _k2_body,
        out_type=[
            jax.ShapeDtypeStruct((_Q * _P,), jnp.float32),
            jax.ShapeDtypeStruct((_Q,), jnp.float32),
        ],
        mesh=_mesh,
        scratch_types=[
            pltpu.VMEM((_NW * 32,), jnp.float32),
            pltpu.VMEM((_NW * 16,), jnp.int32),
            pltpu.VMEM((_TAILW,), jnp.float32),
            pltpu.VMEM((16,), jnp.float32),
            pltpu.VMEM((16,), jnp.int32),
            pltpu.SemaphoreType.DMA,
        ],
        compiler_params=_params,
        name="tnn_k2",
    )
    stdp, li = k2(pf, pi)
    return inp_out, stdp, li


def kernel(data, weights):
    inp12 = jnp.tile(data.astype(jnp.float32).reshape(-1), 3)
    inp16 = jnp.concatenate([inp12, jnp.full((4,), _INF, jnp.float32)])
    wflat = weights.astype(jnp.float32).reshape(-1)
    inp_out, stdp, li = _run(inp16, wflat)
    return (li.reshape(1, 1, _Q),
            inp_out.reshape(_Q, _P),
            stdp.reshape(_Q, _P))


# speedup vs baseline: 2.3686x; 1.0571x over previous
import jax
import jax.numpy as jnp
from jax import lax
from jax.experimental import pallas as pl
from jax.experimental.pallas import tpu as pltpu
from jax.experimental.pallas import tpu_sc as plsc

_Q = 250000
_P = 12
_NWMAX = 7
_THETA = 20.0
_NW = 32
_NG = 489
_RPT = _NG * 16
_LAST = _Q - _RPT
_TAILROWS = _Q - 31 * _RPT
_FILLW = _RPT * _P
_TAILW = _TAILROWS * _P
_EXTW = _FILLW - _TAILW
_INF = float("inf")

_mesh = plsc.VectorSubcoreMesh(core_axis_name="c", subcore_axis_name="s")
_params = pltpu.CompilerParams(needs_layout_passes=False)


def _wid():
    return lax.axis_index("c") * 16 + lax.axis_index("s")


def _splat(x, dtype=jnp.float32):
    return jnp.full((16,), x, dtype=dtype)


def _k1_body(inp16_hbm, w_hbm, inpout_hbm, pf_hbm, pi_hbm,
             vm_inp, vm_perm, vm_w, vm_fill, vm_pf, vm_pi,
             sem_w, sem_f):
    wid = _wid()
    start = jnp.minimum(wid * _RPT, _LAST)
    wofs = pl.multiple_of(start * _P, 8)
    wcopy = pltpu.make_async_copy(w_hbm.at[pl.ds(wofs, _FILLW)], vm_w, sem_w)
    wcopy.start()

    pltpu.sync_copy(inp16_hbm, vm_inp)
    v = vm_inp[...]
    iota = lax.iota(jnp.int32, 16)
    one = jnp.ones((16,), jnp.int32)
    zero = jnp.zeros((16,), jnp.int32)
    r = zero
    for k in range(_P):
        vk = _splat(v[k])
        r = r + jnp.where(vk < v, one, zero)
        r = r + jnp.where((vk == v) & (iota > k), one, zero)
    r = jnp.where(iota < _P, r, iota)
    plsc.store_scatter(vm_perm, [r], iota)
    pv = vm_perm[...]
    st = plsc.load_gather(vm_inp, [pv])

    pat = [plsc.load_gather(vm_inp, [(iota + 16 * k) % _P]) for k in range(3)]

    def fill_body(j, c):
        base = j * 48
        vm_fill[pl.ds(base, 16)] = pat[0]
        vm_fill[pl.ds(base + 16, 16)] = pat[1]
        vm_fill[pl.ds(base + 32, 16)] = pat[2]
        return c

    lax.fori_loop(0, 256, fill_body, 0)

    fills = []
    for k in range(8):
        s_k = min(k * 1024, _RPT - 1024)
        ofs = pl.multiple_of((start + s_k) * _P, 8)
        c = pltpu.make_async_copy(vm_fill, inpout_hbm.at[pl.ds(ofs, 12288)], sem_f)
        c.start()
        fills.append(c)

    perm_s = [pv[p] for p in range(_P)]
    st_s = [st[p] for p in range(_P)]
    c1v = [_splat(1.0 - st_s[p]) for p in range(_P)]
    emask = [_splat(st_s[p]) == _INF for p in range(_P)]
    liota12 = iota * _P

    wcopy.wait()

    def group(g, carry):
        m, s, i = carry
        b = g * (16 * _P)
        wr = [plsc.load_gather(vm_w, [liota12 + (b + perm_s[p])])
              for p in range(_P)]
        acc = jnp.where(emask[0], 0.0, wr[0])
        acc0 = acc
        found = acc >= _THETA
        ec = jnp.where(found, _splat(st_s[0]), _INF)
        pot = jnp.where(found, acc, 0.0)
        for p in range(1, _P):
            acc = acc + jnp.where(emask[p], 0.0, wr[p])
            cross = (acc >= _THETA) & (~found)
            ec = jnp.where(cross, _splat(st_s[p]), ec)
            pot = jnp.where(cross, acc, pot)
            found = found | cross
        u = [ec + c1v[p] for p in range(_P)]

        def rsum(wf):
            t0 = jnp.maximum(jnp.minimum(u[0] + wf, wr[0]), 0.0)
            t1 = jnp.maximum(jnp.minimum(u[1] + wf, wr[1]), 0.0)
            t2 = jnp.maximum(jnp.minimum(u[2] + wf, wr[2]), 0.0)
            for p in range(3, _P, 3):
                t0 = t0 + jnp.maximum(jnp.minimum(u[p] + wf, wr[p]), 0.0)
                t1 = t1 + jnp.maximum(jnp.minimum(u[p + 1] + wf, wr[p + 1]), 0.0)
                t2 = t2 + jnp.maximum(jnp.minimum(u[p + 2] + wf, wr[p + 2]), 0.0)
            return t0 + t1 + t2

        lo = jnp.zeros((16,), jnp.int32)
        hi = _splat(_NWMAX, jnp.int32)
        for _ in range(3):
            mid = (lo + hi) >> 1
            ok = rsum(mid.astype(jnp.float32)) >= _THETA
            hi = jnp.where(ok, mid, hi)
            lo = jnp.where(ok, lo, mid + 1)
        ind2 = jnp.where(lo < _NWMAX, lo, 0).astype(jnp.float32)
        potr = rsum(ind2)
        ec_f = jnp.where(found, ec + ind2, _INF)
        pot_f = jnp.where(found, potr, acc0)
        row = iota + (start + g * 16)
        lt = ec_f < m
        upd = lt | ((ec_f == m) & (pot_f > s))
        m = jnp.where(lt, ec_f, m)
        s = jnp.where(upd, pot_f, s)
        i = jnp.where(upd, row, i)
        return m, s, i

    init = (_splat(_INF), _splat(-_INF), jnp.zeros((16,), jnp.int32))
    m, s, i = lax.fori_loop(0, _NG, group, init)

    vm_pf[pl.ds(0, 16)] = m
    vm_pf[pl.ds(16, 16)] = s
    vm_pi[...] = i
    pltpu.sync_copy(vm_pf, pf_hbm.at[pl.ds(pl.multiple_of(wid * 32, 8), 32)])
    pltpu.sync_copy(vm_pi, pi_hbm.at[pl.ds(pl.multiple_of(wid * 16, 8), 16)])
    for c in fills:
        c.wait()


def _k2_body(pf_hbm, pi_hbm, stdp_hbm, next_hbm,
             vm_pf, vm_pi, vm_inf, vm_win, vm_idx, sem):
    wid = _wid()
    pltpu.sync_copy(pf_hbm, vm_pf)
    pltpu.sync_copy(pi_hbm, vm_pi)

    m = vm_pf[pl.ds(0, 16)]
    s = vm_pf[pl.ds(16, 16)]
    i = vm_pi[pl.ds(0, 16)]
    for t in range(1, _NW):
        mt = vm_pf[pl.ds(t * 32, 16)]
        st_ = vm_pf[pl.ds(t * 32 + 16, 16)]
        it = vm_pi[pl.ds(t * 16, 16)]
        take = (mt < m) | ((mt == m) & ((st_ > s) | ((st_ == s) & (it < i))))
        m = jnp.where(take, mt, m)
        s = jnp.where(take, st_, s)
        i = jnp.where(take, it, i)
    minv = jnp.min(m)
    mask = m == minv
    sm = jnp.where(mask, s, -_INF)
    smax = jnp.max(sm)
    ibig = jnp.where(mask & (sm == smax), i, jnp.int32(2**31 - 1))
    iid = jnp.min(ibig)

    infv = _splat(_INF)

    def fill_body(j, c):
        vm_inf[pl.ds(j * 16, 16)] = infv
        return c

    lax.fori_loop(0, _TAILW // 16, fill_body, 0)

    start = wid * _RPT
    pltpu.sync_copy(vm_inf,
                    stdp_hbm.at[pl.ds(pl.multiple_of(start * _P, 8), _TAILW)])
    pltpu.sync_copy(vm_inf.at[pl.ds(0, _TAILROWS)],
                    next_hbm.at[pl.ds(pl.multiple_of(start, 8), _TAILROWS)])

    @pl.when(wid < _NW - 1)
    def _extra():
        pltpu.sync_copy(
            vm_inf.at[pl.ds(0, _EXTW)],
            stdp_hbm.at[pl.ds(pl.multiple_of(start * _P + _TAILW, 8), _EXTW)])
        pltpu.sync_copy(
            vm_inf.at[pl.ds(0, _RPT - _TAILROWS)],
            next_hbm.at[pl.ds(pl.multiple_of(start + _TAILROWS, 8),
                              _RPT - _TAILROWS)])

    span = jnp.minimum(start + _RPT, _Q)

    @pl.when((iid >= start) & (iid < span))
    def _scatter():
        vm_win[...] = _splat(minv)
        offs = jnp.where(lax.iota(jnp.int32, 16) < _P,
                         lax.iota(jnp.int32, 16), 0)
        vm_idx[...] = offs + iid * _P
        pltpu.async_copy(vm_win, stdp_hbm.at[vm_idx], sem).wait()
        vm_idx[...] = _splat(iid, jnp.int32)
        pltpu.async_copy(vm_win, next_hbm.at[vm_idx], sem).wait()


@jax.jit
def _run(inp16, wflat):
    k1 = pl.kernel(
        _k1_body,
        out_type=[
            jax.ShapeDtypeStruct((_Q * _P,), jnp.float32),
            jax.ShapeDtypeStruct((_NW * 32,), jnp.float32),
            jax.ShapeDtypeStruct((_NW * 16,), jnp.int32),
        ],
        mesh=_mesh,
        scratch_types=[
            pltpu.VMEM((16,), jnp.float32),
            pltpu.VMEM((16,), jnp.int32),
            pltpu.VMEM((_FILLW,), jnp.float32),
            pltpu.VMEM((12288,), jnp.float32),
            pltpu.VMEM((32,), jnp.float32),
            pltpu.VMEM((16,), jnp.int32),
            pltpu.SemaphoreType.DMA,
            pltpu.SemaphoreType.DMA,
        ],
        compiler_params=_params,
        name="tnn_k1",
    )
    inp_out, pf, pi = k1(inp16, wflat)
    k2 = pl.kernel(
        _k2_body,
        out_type=[
            jax.ShapeDtypeStruct((_Q * _P,), jnp.float32),
            jax.ShapeDtypeStruct((_Q,), jnp.float32),
        ],
        mesh=_mesh,
        scratch_types=[
            pltpu.VMEM((_NW * 32,), jnp.float32),
            pltpu.VMEM((_NW * 16,), jnp.int32),
            pltpu.VMEM((_TAILW,), jnp.float32),
            pltpu.VMEM((16,), jnp.float32),
            pltpu.VMEM((16,), jnp.int32),
            pltpu.SemaphoreType.DMA,
        ],
        compiler_params=_params,
        name="tnn_k2",
    )
    stdp, li = k2(pf, pi)
    return inp_out, stdp, li


def kernel(data, weights):
    inp12 = jnp.tile(data.astype(jnp.float32).reshape(-1), 3)
    inp16 = jnp.concatenate([inp12, jnp.full((4,), _INF, jnp.float32)])
    wflat = weights.astype(jnp.float32).reshape(-1)
    inp_out, stdp, li = _run(inp16, wflat)
    return (li.reshape(1, 1, _Q),
            inp_out.reshape(_Q, _P),
            stdp.reshape(_Q, _P))


# speedup vs baseline: 3.6854x; 1.5559x over previous
import jax
import jax.numpy as jnp
from jax import lax
from jax.experimental import pallas as pl
from jax.experimental.pallas import tpu as pltpu
from jax.experimental.pallas import tpu_sc as plsc

_Q = 250000
_P = 12
_NWMAX = 7
_THETA = 20.0
_NW = 32
_NG = 489
_RPT = _NG * 16
_LAST = _Q - _RPT
_INF = float("inf")
_BQ = 2048

_mesh = plsc.VectorSubcoreMesh(core_axis_name="c", subcore_axis_name="s")
_params = pltpu.CompilerParams(needs_layout_passes=False,
                               use_tc_tiling_on_sc=False)


def _wid():
    return lax.axis_index("c") * 16 + lax.axis_index("s")


def _splat(x, dtype=jnp.float32):
    return jnp.full((16,), x, dtype=dtype)


def _k1_body(inp16_hbm, w_hbm, pm_hbm, ps_hbm, pi_hbm,
             vm_inp, vm_perm, vm_w0, vm_w1, vm_pm, vm_ps, vm_pi,
             sem_w0, sem_w1):
    wid = _wid()
    start = jnp.minimum(wid * _RPT, _LAST)
    cstarts = [min(c * 2048, _RPT - 2048) for c in range(4)]
    wbufs = [vm_w0, vm_w1]
    wsems = [sem_w0, sem_w1]

    def wdma(c):
        return pltpu.make_async_copy(
            w_hbm.at[pl.ds(start + cstarts[c], 2048), :],
            wbufs[c % 2], wsems[c % 2])

    wdma(0).start()
    wdma(1).start()

    pltpu.sync_copy(inp16_hbm, vm_inp)
    v = vm_inp[...]
    iota = lax.iota(jnp.int32, 16)
    one = jnp.ones((16,), jnp.int32)
    zero = jnp.zeros((16,), jnp.int32)
    r = zero
    for k in range(_P):
        vk = _splat(v[k])
        r = r + jnp.where(vk < v, one, zero)
        r = r + jnp.where((vk == v) & (iota > k), one, zero)
    r = jnp.where(iota < _P, r, iota)
    plsc.store_scatter(vm_perm, [r], iota)
    pv = vm_perm[...]
    st = plsc.load_gather(vm_inp, [pv])

    perm_s = [pv[p] for p in range(_P)]
    st_s = [st[p] for p in range(_P)]
    c1v = [_splat(1.0 - st_s[p]) for p in range(_P)]
    emask = [_splat(st_s[p]) == _INF for p in range(_P)]
    colp = [_splat(perm_s[p], jnp.int32) for p in range(_P)]

    def compute(vm_w, g):
        rowg = iota + g * 16
        wr = [plsc.load_gather(vm_w, [rowg, colp[p]]) for p in range(_P)]
        acc = jnp.where(emask[0], 0.0, wr[0])
        acc0 = acc
        found = acc >= _THETA
        ec = jnp.where(found, _splat(st_s[0]), _INF)
        pot = jnp.where(found, acc, 0.0)
        for p in range(1, _P):
            acc = acc + jnp.where(emask[p], 0.0, wr[p])
            cross = (acc >= _THETA) & (~found)
            ec = jnp.where(cross, _splat(st_s[p]), ec)
            pot = jnp.where(cross, acc, pot)
            found = found | cross
        u = [ec + c1v[p] for p in range(_P)]

        def rsum(wf):
            t0 = jnp.maximum(jnp.minimum(u[0] + wf, wr[0]), 0.0)
            t1 = jnp.maximum(jnp.minimum(u[1] + wf, wr[1]), 0.0)
            t2 = jnp.maximum(jnp.minimum(u[2] + wf, wr[2]), 0.0)
            for p in range(3, _P, 3):
                t0 = t0 + jnp.maximum(jnp.minimum(u[p] + wf, wr[p]), 0.0)
                t1 = t1 + jnp.maximum(jnp.minimum(u[p + 1] + wf, wr[p + 1]), 0.0)
                t2 = t2 + jnp.maximum(jnp.minimum(u[p + 2] + wf, wr[p + 2]), 0.0)
            return t0 + t1 + t2

        lo = jnp.zeros((16,), jnp.int32)
        hi = _splat(_NWMAX, jnp.int32)
        for _ in range(3):
            mid = (lo + hi) >> 1
            ok = rsum(mid.astype(jnp.float32)) >= _THETA
            hi = jnp.where(ok, mid, hi)
            lo = jnp.where(ok, lo, mid + 1)
        ind2 = jnp.where(lo < _NWMAX, lo, 0).astype(jnp.float32)
        potr = rsum(ind2)
        ec_f = jnp.where(found, ec + ind2, _INF)
        pot_f = jnp.where(found, potr, acc0)
        return ec_f, pot_f

    def update(rowbase, g, ec_f, pot_f, carry):
        m, s, i = carry
        row = iota + (rowbase + g * 16)
        lt = ec_f < m
        upd = lt | ((ec_f == m) & (pot_f > s))
        m = jnp.where(lt, ec_f, m)
        s = jnp.where(upd, pot_f, s)
        i = jnp.where(upd, row, i)
        return m, s, i

    carry = (_splat(_INF), _splat(-_INF), jnp.zeros((16,), jnp.int32))
    for c in range(4):
        wdma(c).wait()
        if c + 2 < 4:
            wdma(c + 2).start()
        vm_w = wbufs[c % 2]
        rowbase = start + cstarts[c]

        def group2(g2, carry, vm_w=vm_w, rowbase=rowbase):
            ga = g2 + g2
            gb = ga + 1
            ra = compute(vm_w, ga)
            rb = compute(vm_w, gb)
            carry = update(rowbase, ga, ra[0], ra[1], carry)
            return update(rowbase, gb, rb[0], rb[1], carry)

        carry = lax.fori_loop(0, 64, group2, carry)
    m, s, i = carry

    vm_pm[...] = m
    vm_ps[...] = s
    vm_pi[...] = i
    ofs = pl.multiple_of(wid * 16, 8)
    pltpu.sync_copy(vm_pm, pm_hbm.at[pl.ds(ofs, 16)])
    pltpu.sync_copy(vm_ps, ps_hbm.at[pl.ds(ofs, 16)])
    pltpu.sync_copy(vm_pi, pi_hbm.at[pl.ds(ofs, 16)])


def _k2_body(inp12_ref, pm_ref, ps_ref, pi_ref,
             inpt_ref, stdpt_ref, next_ref):
    m = pm_ref[...]
    s = ps_ref[...]
    i = pi_ref[...]
    minv = jnp.min(m)
    mask = m == minv
    sm = jnp.where(mask, s, -_INF)
    smax = jnp.max(sm)
    ibig = jnp.where(mask & (sm == smax), i, jnp.int32(2**31 - 1))
    iid = jnp.min(ibig)

    j = pl.program_id(0)
    col1 = j * _BQ + lax.broadcasted_iota(jnp.int32, (1, _BQ), 1)
    next_ref[...] = jnp.where(col1 == iid, minv, _INF).reshape(1, 1, _BQ)
    colp = j * _BQ + lax.broadcasted_iota(jnp.int32, (_P, _BQ), 1)
    stdpt_ref[...] = jnp.where(colp == iid, minv, _INF)
    inpt_ref[...] = jnp.broadcast_to(inp12_ref[...], (_P, _BQ))


@jax.jit
def _run(inp16, inp12c, weights):
    k1 = pl.kernel(
        _k1_body,
        out_type=[
            jax.ShapeDtypeStruct((_NW * 16,), jnp.float32),
            jax.ShapeDtypeStruct((_NW * 16,), jnp.float32),
            jax.ShapeDtypeStruct((_NW * 16,), jnp.int32),
        ],
        mesh=_mesh,
        scratch_types=[
            pltpu.VMEM((16,), jnp.float32),
            pltpu.VMEM((16,), jnp.int32),
            pltpu.VMEM((2048, _P), jnp.float32),
            pltpu.VMEM((2048, _P), jnp.float32),
            pltpu.VMEM((16,), jnp.float32),
            pltpu.VMEM((16,), jnp.float32),
            pltpu.VMEM((16,), jnp.int32),
            pltpu.SemaphoreType.DMA,
            pltpu.SemaphoreType.DMA,
        ],
        compiler_params=_params,
        name="tnn_k1",
    )
    pm, ps, pi = k1(inp16, weights)
    grid = (_Q + _BQ - 1) // _BQ
    full = pl.BlockSpec((_NW, 16), lambda j: (0, 0))
    inpt, stdpt, nxt = pl.pallas_call(
        _k2_body,
        grid=(grid,),
        in_specs=[
            pl.BlockSpec((_P, 1), lambda j: (0, 0)),
            full, full, full,
        ],
        out_specs=[
            pl.BlockSpec((_P, _BQ), lambda j: (0, j)),
            pl.BlockSpec((_P, _BQ), lambda j: (0, j)),
            pl.BlockSpec((1, 1, _BQ), lambda j: (0, 0, j)),
        ],
        out_shape=[
            jax.ShapeDtypeStruct((_P, _Q), jnp.float32),
            jax.ShapeDtypeStruct((_P, _Q), jnp.float32),
            jax.ShapeDtypeStruct((1, 1, _Q), jnp.float32),
        ],
        name="tnn_k2",
    )(inp12c, pm.reshape(_NW, 16), ps.reshape(_NW, 16), pi.reshape(_NW, 16))
    return inpt, stdpt, nxt


def kernel(data, weights):
    inp12 = jnp.tile(data.astype(jnp.float32).reshape(-1), 3)
    inp16 = jnp.concatenate([inp12, jnp.full((4,), _INF, jnp.float32)])
    inpt, stdpt, nxt = _run(inp16, inp12.reshape(_P, 1), weights)
    return (nxt, inpt.T, stdpt.T)


# speedup vs baseline: 3.7083x; 1.0062x over previous
import jax
import jax.numpy as jnp
from jax import lax
from jax.experimental import pallas as pl
from jax.experimental.pallas import tpu as pltpu
from jax.experimental.pallas import tpu_sc as plsc

_Q = 250000
_P = 12
_NWMAX = 7
_THETA = 20.0
_NW = 32
_NG = 489
_RPT = _NG * 16
_LAST = _Q - _RPT
_INF = float("inf")
_BQ = 2048

_mesh = plsc.VectorSubcoreMesh(core_axis_name="c", subcore_axis_name="s")
_params = pltpu.CompilerParams(needs_layout_passes=False,
                               use_tc_tiling_on_sc=False)


def _wid():
    return lax.axis_index("c") * 16 + lax.axis_index("s")


def _splat(x, dtype=jnp.float32):
    return jnp.full((16,), x, dtype=dtype)


def _k1_body(inp16_hbm, w_hbm, pm_hbm, ps_hbm, pi_hbm,
             vm_inp, vm_perm, vm_w0, vm_w1, vm_pm, vm_ps, vm_pi,
             sem_w0, sem_w1):
    wid = _wid()
    start = jnp.minimum(wid * _RPT, _LAST)
    cstarts = [min(c * 2048, _RPT - 2048) for c in range(4)]
    wbufs = [vm_w0, vm_w1]
    wsems = [sem_w0, sem_w1]

    def wdma(c):
        return pltpu.make_async_copy(
            w_hbm.at[pl.ds(start + cstarts[c], 2048), :],
            wbufs[c % 2], wsems[c % 2])

    wdma(0).start()
    wdma(1).start()

    pltpu.sync_copy(inp16_hbm, vm_inp)
    v = vm_inp[...]
    iota = lax.iota(jnp.int32, 16)
    one = jnp.ones((16,), jnp.int32)
    zero = jnp.zeros((16,), jnp.int32)
    r = zero
    for k in range(_P):
        vk = _splat(v[k])
        r = r + jnp.where(vk < v, one, zero)
        r = r + jnp.where((vk == v) & (iota > k), one, zero)
    r = jnp.where(iota < _P, r, iota)
    plsc.store_scatter(vm_perm, [r], iota)
    pv = vm_perm[...]
    st = plsc.load_gather(vm_inp, [pv])

    perm_s = [pv[p] for p in range(_P)]
    st_s = [st[p] for p in range(_P)]
    c1v = [_splat(1.0 - st_s[p]) for p in range(_P)]
    emask = [_splat(st_s[p]) == _INF for p in range(_P)]
    colp = [_splat(perm_s[p], jnp.int32) for p in range(_P)]

    def compute(vm_w, g):
        rowg = iota + g * 16
        wr = [plsc.load_gather(vm_w, [rowg, colp[p]]) for p in range(_P)]
        acc = jnp.where(emask[0], 0.0, wr[0])
        acc0 = acc
        found = acc >= _THETA
        ec = jnp.where(found, _splat(st_s[0]), _INF)
        pot = jnp.where(found, acc, 0.0)
        for p in range(1, _P):
            acc = acc + jnp.where(emask[p], 0.0, wr[p])
            cross = (acc >= _THETA) & (~found)
            ec = jnp.where(cross, _splat(st_s[p]), ec)
            pot = jnp.where(cross, acc, pot)
            found = found | cross
        u = [ec + c1v[p] for p in range(_P)]

        def rsum(wf):
            t0 = jnp.maximum(jnp.minimum(u[0] + wf, wr[0]), 0.0)
            t1 = jnp.maximum(jnp.minimum(u[1] + wf, wr[1]), 0.0)
            t2 = jnp.maximum(jnp.minimum(u[2] + wf, wr[2]), 0.0)
            for p in range(3, _P, 3):
                t0 = t0 + jnp.maximum(jnp.minimum(u[p] + wf, wr[p]), 0.0)
                t1 = t1 + jnp.maximum(jnp.minimum(u[p + 1] + wf, wr[p + 1]), 0.0)
                t2 = t2 + jnp.maximum(jnp.minimum(u[p + 2] + wf, wr[p + 2]), 0.0)
            return t0 + t1 + t2

        lo = jnp.zeros((16,), jnp.int32)
        hi = _splat(_NWMAX, jnp.int32)
        for _ in range(3):
            mid = (lo + hi) >> 1
            ok = rsum(mid.astype(jnp.float32)) >= _THETA
            hi = jnp.where(ok, mid, hi)
            lo = jnp.where(ok, lo, mid + 1)
        ind2 = jnp.where(lo < _NWMAX, lo, 0).astype(jnp.float32)
        potr = rsum(ind2)
        ec_f = jnp.where(found, ec + ind2, _INF)
        pot_f = jnp.where(found, potr, acc0)
        return ec_f, pot_f

    def update(rowbase, g, ec_f, pot_f, carry):
        m, s, i = carry
        row = iota + (rowbase + g * 16)
        lt = ec_f < m
        upd = lt | ((ec_f == m) & (pot_f > s))
        m = jnp.where(lt, ec_f, m)
        s = jnp.where(upd, pot_f, s)
        i = jnp.where(upd, row, i)
        return m, s, i

    carry = (_splat(_INF), _splat(-_INF), jnp.zeros((16,), jnp.int32))
    for c in range(4):
        wdma(c).wait()
        if c + 2 < 4:
            wdma(c + 2).start()
        vm_w = wbufs[c % 2]
        rowbase = start + cstarts[c]

        def group(g, carry, vm_w=vm_w, rowbase=rowbase):
            ec_f, pot_f = compute(vm_w, g)
            return update(rowbase, g, ec_f, pot_f, carry)

        carry = lax.fori_loop(0, 128, group, carry)
    m, s, i = carry

    vm_pm[...] = m
    vm_ps[...] = s
    vm_pi[...] = i
    ofs = pl.multiple_of(wid * 16, 8)
    pltpu.sync_copy(vm_pm, pm_hbm.at[pl.ds(ofs, 16)])
    pltpu.sync_copy(vm_ps, ps_hbm.at[pl.ds(ofs, 16)])
    pltpu.sync_copy(vm_pi, pi_hbm.at[pl.ds(ofs, 16)])


def _k2_body(inp12_ref, pm_ref, ps_ref, pi_ref,
             inpt_ref, stdpt_ref, next_ref):
    m = pm_ref[...]
    s = ps_ref[...]
    i = pi_ref[...]
    minv = jnp.min(m)
    mask = m == minv
    sm = jnp.where(mask, s, -_INF)
    smax = jnp.max(sm)
    ibig = jnp.where(mask & (sm == smax), i, jnp.int32(2**31 - 1))
    iid = jnp.min(ibig)

    j = pl.program_id(0)
    col1 = j * _BQ + lax.broadcasted_iota(jnp.int32, (1, _BQ), 1)
    next_ref[...] = jnp.where(col1 == iid, minv, _INF).reshape(1, 1, _BQ)
    colp = j * _BQ + lax.broadcasted_iota(jnp.int32, (_P, _BQ), 1)
    stdpt_ref[...] = jnp.where(colp == iid, minv, _INF)
    inpt_ref[...] = jnp.broadcast_to(inp12_ref[...], (_P, _BQ))


@jax.jit
def _run(inp16, inp12c, weights):
    k1 = pl.kernel(
        _k1_body,
        out_type=[
            jax.ShapeDtypeStruct((_NW * 16,), jnp.float32),
            jax.ShapeDtypeStruct((_NW * 16,), jnp.float32),
            jax.ShapeDtypeStruct((_NW * 16,), jnp.int32),
        ],
        mesh=_mesh,
        scratch_types=[
            pltpu.VMEM((16,), jnp.float32),
            pltpu.VMEM((16,), jnp.int32),
            pltpu.VMEM((2048, _P), jnp.float32),
            pltpu.VMEM((2048, _P), jnp.float32),
            pltpu.VMEM((16,), jnp.float32),
            pltpu.VMEM((16,), jnp.float32),
            pltpu.VMEM((16,), jnp.int32),
            pltpu.SemaphoreType.DMA,
            pltpu.SemaphoreType.DMA,
        ],
        compiler_params=_params,
        name="tnn_k1",
    )
    pm, ps, pi = k1(inp16, weights)
    grid = (_Q + _BQ - 1) // _BQ
    full = pl.BlockSpec((_NW, 16), lambda j: (0, 0))
    inpt, stdpt, nxt = pl.pallas_call(
        _k2_body,
        grid=(grid,),
        in_specs=[
            pl.BlockSpec((_P, 1), lambda j: (0, 0)),
            full, full, full,
        ],
        out_specs=[
            pl.BlockSpec((_P, _BQ), lambda j: (0, j)),
            pl.BlockSpec((_P, _BQ), lambda j: (0, j)),
            pl.BlockSpec((1, 1, _BQ), lambda j: (0, 0, j)),
        ],
        out_shape=[
            jax.ShapeDtypeStruct((_P, _Q), jnp.float32),
            jax.ShapeDtypeStruct((_P, _Q), jnp.float32),
            jax.ShapeDtypeStruct((1, 1, _Q), jnp.float32),
        ],
        name="tnn_k2",
    )(inp12c, pm.reshape(_NW, 16), ps.reshape(_NW, 16), pi.reshape(_NW, 16))
    return inpt, stdpt, nxt


def kernel(data, weights):
    inp12 = jnp.tile(data.astype(jnp.float32).reshape(-1), 3)
    inp16 = jnp.concatenate([inp12, jnp.full((4,), _INF, jnp.float32)])
    inpt, stdpt, nxt = _run(inp16, inp12.reshape(_P, 1), weights)
    return (nxt, inpt.T, stdpt.T)


# speedup vs baseline: 4.5022x; 1.2141x over previous
import jax
import jax.numpy as jnp
from jax import lax
from jax.experimental import pallas as pl
from jax.experimental.pallas import tpu as pltpu
from jax.experimental.pallas import tpu_sc as plsc

_Q = 250000
_P = 12
_NWMAX = 7
_THETA = 20.0
_NW = 32
_NG = 489
_RPT = _NG * 16
_LAST = _Q - _RPT
_INF = float("inf")
_BQ = 2048

_mesh = plsc.VectorSubcoreMesh(core_axis_name="c", subcore_axis_name="s")
_params = pltpu.CompilerParams(needs_layout_passes=False,
                               use_tc_tiling_on_sc=False)


def _wid():
    return lax.axis_index("c") * 16 + lax.axis_index("s")


def _splat(x, dtype=jnp.float32):
    return jnp.full((16,), x, dtype=dtype)


def _k1_body(inp16_hbm, w_hbm, pm_hbm, ps_hbm, pi_hbm,
             vm_inp, vm_perm, vm_w, vm_pm, vm_ps, vm_pi, sem_w):
    wid = _wid()
    start = jnp.minimum(wid * _RPT, _LAST)
    wofs = pl.multiple_of(start * _P, 8)
    wcopy = pltpu.make_async_copy(
        w_hbm.at[pl.ds(wofs, _RPT * _P)], vm_w, sem_w)
    wcopy.start()

    pltpu.sync_copy(inp16_hbm, vm_inp)
    v = vm_inp[...]
    iota = lax.iota(jnp.int32, 16)
    one = jnp.ones((16,), jnp.int32)
    zero = jnp.zeros((16,), jnp.int32)
    r = zero
    for k in range(_P):
        vk = _splat(v[k])
        r = r + jnp.where(vk < v, one, zero)
        r = r + jnp.where((vk == v) & (iota > k), one, zero)
    r = jnp.where(iota < _P, r, iota)
    plsc.store_scatter(vm_perm, [r], iota)
    pv = vm_perm[...]
    st = plsc.load_gather(vm_inp, [pv])

    perm_s = [pv[p] for p in range(_P)]
    st_s = [st[p] for p in range(_P)]
    c1v = [_splat(1.0 - st_s[p]) for p in range(_P)]
    emask = [_splat(st_s[p]) == _INF for p in range(_P)]
    liota12 = iota * _P

    wcopy.wait()

    def compute(g):
        b = g * (16 * _P)
        wr = [plsc.load_gather(vm_w, [liota12 + (b + perm_s[p])])
              for p in range(_P)]
        acc = jnp.where(emask[0], 0.0, wr[0])
        acc0 = acc
        found = acc >= _THETA
        ec = jnp.where(found, _splat(st_s[0]), _INF)
        pot = jnp.where(found, acc, 0.0)
        for p in range(1, _P):
            acc = acc + jnp.where(emask[p], 0.0, wr[p])
            cross = (acc >= _THETA) & (~found)
            ec = jnp.where(cross, _splat(st_s[p]), ec)
            pot = jnp.where(cross, acc, pot)
            found = found | cross
        u = [ec + c1v[p] for p in range(_P)]

        def rsum(wf):
            t0 = jnp.maximum(jnp.minimum(u[0] + wf, wr[0]), 0.0)
            t1 = jnp.maximum(jnp.minimum(u[1] + wf, wr[1]), 0.0)
            t2 = jnp.maximum(jnp.minimum(u[2] + wf, wr[2]), 0.0)
            for p in range(3, _P, 3):
                t0 = t0 + jnp.maximum(jnp.minimum(u[p] + wf, wr[p]), 0.0)
                t1 = t1 + jnp.maximum(jnp.minimum(u[p + 1] + wf, wr[p + 1]), 0.0)
                t2 = t2 + jnp.maximum(jnp.minimum(u[p + 2] + wf, wr[p + 2]), 0.0)
            return t0 + t1 + t2

        lo = jnp.zeros((16,), jnp.int32)
        hi = _splat(_NWMAX, jnp.int32)
        for _ in range(3):
            mid = (lo + hi) >> 1
            ok = rsum(mid.astype(jnp.float32)) >= _THETA
            hi = jnp.where(ok, mid, hi)
            lo = jnp.where(ok, lo, mid + 1)
        ind2 = jnp.where(lo < _NWMAX, lo, 0).astype(jnp.float32)
        potr = rsum(ind2)
        ec_f = jnp.where(found, ec + ind2, _INF)
        pot_f = jnp.where(found, potr, acc0)
        return ec_f, pot_f

    def update(rowbase, g, ec_f, pot_f, carry):
        m, s, i = carry
        row = iota + (rowbase + g * 16)
        lt = ec_f < m
        upd = lt | ((ec_f == m) & (pot_f > s))
        m = jnp.where(lt, ec_f, m)
        s = jnp.where(upd, pot_f, s)
        i = jnp.where(upd, row, i)
        return m, s, i

    def group(g, carry):
        ec_f, pot_f = compute(g)
        return update(start, g, ec_f, pot_f, carry)

    init = (_splat(_INF), _splat(-_INF), jnp.zeros((16,), jnp.int32))
    m, s, i = lax.fori_loop(0, _NG, group, init)

    vm_pm[...] = m
    vm_ps[...] = s
    vm_pi[...] = i
    ofs = pl.multiple_of(wid * 16, 8)
    pltpu.sync_copy(vm_pm, pm_hbm.at[pl.ds(ofs, 16)])
    pltpu.sync_copy(vm_ps, ps_hbm.at[pl.ds(ofs, 16)])
    pltpu.sync_copy(vm_pi, pi_hbm.at[pl.ds(ofs, 16)])


def _k2_body(inp12_ref, pm_ref, ps_ref, pi_ref,
             inpt_ref, stdpt_ref, next_ref):
    m = pm_ref[...]
    s = ps_ref[...]
    i = pi_ref[...]
    minv = jnp.min(m)
    mask = m == minv
    sm = jnp.where(mask, s, -_INF)
    smax = jnp.max(sm)
    ibig = jnp.where(mask & (sm == smax), i, jnp.int32(2**31 - 1))
    iid = jnp.min(ibig)

    j = pl.program_id(0)
    col1 = j * _BQ + lax.broadcasted_iota(jnp.int32, (1, _BQ), 1)
    next_ref[...] = jnp.where(col1 == iid, minv, _INF).reshape(1, 1, _BQ)
    colp = j * _BQ + lax.broadcasted_iota(jnp.int32, (_P, _BQ), 1)
    stdpt_ref[...] = jnp.where(colp == iid, minv, _INF)
    inpt_ref[...] = jnp.broadcast_to(inp12_ref[...], (_P, _BQ))


@jax.jit
def _run(inp16, inp12c, weights):
    k1 = pl.kernel(
        _k1_body,
        out_type=[
            jax.ShapeDtypeStruct((_NW * 16,), jnp.float32),
            jax.ShapeDtypeStruct((_NW * 16,), jnp.float32),
            jax.ShapeDtypeStruct((_NW * 16,), jnp.int32),
        ],
        mesh=_mesh,
        scratch_types=[
            pltpu.VMEM((16,), jnp.float32),
            pltpu.VMEM((16,), jnp.int32),
            pltpu.VMEM((_RPT * _P,), jnp.float32),
            pltpu.VMEM((16,), jnp.float32),
            pltpu.VMEM((16,), jnp.float32),
            pltpu.VMEM((16,), jnp.int32),
            pltpu.SemaphoreType.DMA,
        ],
        compiler_params=_params,
        name="tnn_k1",
    )
    pm, ps, pi = k1(inp16, weights.reshape(-1))
    grid = (_Q + _BQ - 1) // _BQ
    full = pl.BlockSpec((_NW, 16), lambda j: (0, 0))
    inpt, stdpt, nxt = pl.pallas_call(
        _k2_body,
        grid=(grid,),
        in_specs=[
            pl.BlockSpec((_P, 1), lambda j: (0, 0)),
            full, full, full,
        ],
        out_specs=[
            pl.BlockSpec((_P, _BQ), lambda j: (0, j)),
            pl.BlockSpec((_P, _BQ), lambda j: (0, j)),
            pl.BlockSpec((1, 1, _BQ), lambda j: (0, 0, j)),
        ],
        out_shape=[
            jax.ShapeDtypeStruct((_P, _Q), jnp.float32),
            jax.ShapeDtypeStruct((_P, _Q), jnp.float32),
            jax.ShapeDtypeStruct((1, 1, _Q), jnp.float32),
        ],
        name="tnn_k2",
    )(inp12c, pm.reshape(_NW, 16), ps.reshape(_NW, 16), pi.reshape(_NW, 16))
    return inpt, stdpt, nxt


def kernel(data, weights):
    inp12 = jnp.tile(data.astype(jnp.float32).reshape(-1), 3)
    inp16 = jnp.concatenate([inp12, jnp.full((4,), _INF, jnp.float32)])
    inpt, stdpt, nxt = _run(inp16, inp12.reshape(_P, 1), weights)
    return (nxt, inpt.T, stdpt.T)
